# baseline (speedup 1.0000x reference)
import functools

import jax
import jax.numpy as jnp
from jax import lax
from jax.experimental import pallas as pl
from jax.experimental.pallas import tpu as pltpu
from jax.experimental.pallas import tpu_sc as plsc

N = 10000
E = 320000
D = 128
C = 2
G = 16

NC = 2
NSUB = 16
NW = NC * NSUB
NBUF = 2

CHD = 128
CHWD = 80
EPADD = NW * CHWD * CHD

CH = 96
CHW = 108
HALF = 2
HCHW = CHW // HALF
EPAD = NW * CHW * CH

NP = 10112
TPT = NP // NSUB
DUMMY = N

ROWS = 1000

def _make_splits(step):
    out, off = [], 0
    while off < TPT:
        out.append((off, min(step, TPT - off)))
        off += step
    return out

_SPLITS = _make_splits(CH)
_SPLITS_D = _make_splits(CHD)


def _sc_mesh():
    return plsc.VectorSubcoreMesh(core_axis_name="c", subcore_axis_name="s")


@functools.partial(
    pl.kernel,
    out_type=jax.ShapeDtypeStruct((NC, NP, D), jnp.float32),
    mesh=_sc_mesh(),
    scratch_types=[
        pltpu.VMEM((CHWD, CHD), jnp.int32),
        pltpu.VMEM((CHD, D), jnp.float32),
        pltpu.VMEM((CHD, D), jnp.float32),
        pltpu.VMEM_SHARED((NP, D), jnp.float32),
    ],
)
def _sc_deg(dstp_hbm, ones_hbm, zeros_hbm, out_hbm, dstw, ones_v, z_v, acc):
    c = lax.axis_index("c")
    s = lax.axis_index("s")
    wid = c * NSUB + s
    pltpu.sync_copy(dstp_hbm.at[wid], dstw)
    pltpu.sync_copy(ones_hbm, ones_v)
    pltpu.sync_copy(zeros_hbm, z_v)
    for off, sz in _SPLITS_D:
        pltpu.sync_copy(z_v.at[pl.ds(0, sz)],
                        acc.at[pl.ds(s * TPT + off, sz)])
    plsc.subcore_barrier()

    def chunk(j, carry):
        pltpu.sync_copy(ones_v, acc.at[dstw.at[j]], add=True)
        return carry

    lax.fori_loop(0, CHWD, chunk, 0)
    plsc.subcore_barrier()
    for off, sz in _SPLITS_D:
        pltpu.sync_copy(acc.at[pl.ds(s * TPT + off, sz)],
                        z_v.at[pl.ds(0, sz)])
        pltpu.sync_copy(z_v.at[pl.ds(0, sz)],
                        out_hbm.at[c, pl.ds(s * TPT + off, sz)])


@functools.partial(
    pl.kernel,
    out_type=jax.ShapeDtypeStruct((NC, NP, D), jnp.float32),
    mesh=_sc_mesh(),
    scratch_types=[
        pltpu.VMEM((2, HCHW, CH), jnp.int32),
        pltpu.VMEM((CH, D), jnp.float32),
        pltpu.VMEM((CH, D), jnp.float32),
        pltpu.VMEM_SHARED((NP, D), jnp.float32),
        pltpu.SemaphoreType.DMA,
        pltpu.SemaphoreType.DMA,
    ],
)
def _sc_agg(g_hbm, idxp_hbm, zrows_hbm, out_hbm,
            idxw, rows0, rows1, acc, sem0, sem1):
    rows = (rows0, rows1)
    sems = (sem0, sem1)
    srcw = idxw.at[0]
    dstw = idxw.at[1]
    c = lax.axis_index("c")
    s = lax.axis_index("s")
    wid = c * NSUB + s
    pltpu.sync_copy(zrows_hbm, rows[0])
    for off, sz in _SPLITS:
        pltpu.sync_copy(rows[0].at[pl.ds(0, sz)],
                        acc.at[pl.ds(s * TPT + off, sz)])
    plsc.subcore_barrier()

    for h in range(HALF):
        pltpu.sync_copy(idxp_hbm.at[wid, h], idxw)
        for b in range(NBUF):
            pltpu.async_copy(g_hbm.at[srcw.at[b]], rows[b], sems[b])

        def body(i, carry):
            base = i * NBUF
            for b in range(NBUF):
                e = base + b
                pltpu.make_async_copy(
                    g_hbm.at[srcw.at[e]], rows[b], sems[b]).wait()
                pltpu.sync_copy(rows[b], acc.at[dstw.at[e]], add=True)
                ne = e + NBUF

                @pl.when(ne < HCHW)
                def _():
                    pltpu.async_copy(
                        g_hbm.at[srcw.at[ne]], rows[b], sems[b])
            return carry

        lax.fori_loop(0, HCHW // NBUF, body, 0)
    plsc.subcore_barrier()
    for off, sz in _SPLITS:
        pltpu.sync_copy(acc.at[pl.ds(s * TPT + off, sz)],
                        rows[0].at[pl.ds(0, sz)])
        pltpu.sync_copy(rows[0].at[pl.ds(0, sz)],
                        out_hbm.at[c, pl.ds(s * TPT + off, sz)])


def _dinv_of(degcol):
    return lax.rsqrt(degcol[0] + degcol[1] + 1.0)


def _m1_body(x_ref, w_ref, degcol_ref, out_ref):
    dinv = _dinv_of(degcol_ref[...])
    out_ref[...] = jnp.dot(x_ref[...], w_ref[...],
                           preferred_element_type=jnp.float32) * dinv


def _mlayer_body(aggp_ref, gprev_ref, degcol_ref, b_ref, w_ref, out_ref):
    dinv = _dinv_of(degcol_ref[...])
    a = dinv * (aggp_ref[0] + aggp_ref[1] + gprev_ref[...]) + b_ref[...]
    a = jnp.maximum(a, 0.0)
    out_ref[...] = jnp.dot(a, w_ref[...],
                           preferred_element_type=jnp.float32) * dinv


_GSPEC = [
    pl.BlockSpec((NC, ROWS, D), lambda i: (0, i, 0)),
    pl.BlockSpec((ROWS, D), lambda i: (i, 0)),
    pl.BlockSpec((NC, ROWS, 1), lambda i: (0, i, 0)),
    pl.BlockSpec((1, D), lambda i: (0, 0)),
    pl.BlockSpec((D, D), lambda i: (0, 0)),
]


def _m1(x, W, degcol):
    return pl.pallas_call(
        _m1_body,
        grid=(N // ROWS,),
        in_specs=[
            pl.BlockSpec((ROWS, D), lambda i: (i, 0)),
            pl.BlockSpec((D, D), lambda i: (0, 0)),
            pl.BlockSpec((NC, ROWS, 1), lambda i: (0, i, 0)),
        ],
        out_specs=pl.BlockSpec((ROWS, D), lambda i: (i, 0)),
        out_shape=jax.ShapeDtypeStruct((N, D), jnp.float32),
    )(x, W, degcol)


def _mlayer(aggp, gprev, degcol, b, W):
    return pl.pallas_call(
        _mlayer_body,
        grid=(N // ROWS,),
        in_specs=_GSPEC,
        out_specs=pl.BlockSpec((ROWS, D), lambda i: (i, 0)),
        out_shape=jax.ShapeDtypeStruct((N, D), jnp.float32),
    )(aggp, gprev, degcol, b.reshape(1, D), W)


def _pool_body(aggp_ref, g_ref, degcol_ref, b_ref, ids_ref, wf_ref, bf_ref,
               out_ref, scr_ref):
    i = pl.program_id(0)

    @pl.when(i == 0)
    def _():
        scr_ref[...] = jnp.full((G, D), -jnp.inf, jnp.float32)

    dinv = _dinv_of(degcol_ref[...])
    post = dinv * (aggp_ref[0] + aggp_ref[1] + g_ref[...]) + b_ref[...]
    ids = ids_ref[...]
    cur = scr_ref[...]
    maxes = jnp.stack(
        [jnp.max(jnp.where(ids == float(g), post, -jnp.inf), axis=0)
         for g in range(G)])
    cur = jnp.maximum(cur, maxes)
    scr_ref[...] = cur

    @pl.when(i == pl.num_programs(0) - 1)
    def _():
        pooled = jnp.where(jnp.isfinite(cur), cur, 0.0)
        out_ref[...] = (
            jnp.dot(pooled, wf_ref[...], preferred_element_type=jnp.float32)
            + bf_ref[...])


def _pool(aggp, g3, degcol, b3, batch, Wf, bf):
    ids = batch.astype(jnp.float32).reshape(N, 1)
    return pl.pallas_call(
        _pool_body,
        grid=(N // ROWS,),
        in_specs=_GSPEC[:4] + [
            pl.BlockSpec((ROWS, 1), lambda i: (i, 0)),
            pl.BlockSpec((D, C), lambda i: (0, 0)),
            pl.BlockSpec((1, C), lambda i: (0, 0)),
        ],
        out_specs=pl.BlockSpec((G, C), lambda i: (0, 0)),
        out_shape=jax.ShapeDtypeStruct((G, C), jnp.float32),
        scratch_shapes=[pltpu.VMEM((G, D), jnp.float32)],
    )(aggp, g3, degcol, b3.reshape(1, D), ids, Wf, bf.reshape(1, C))


def _padded(v, total, fill):
    return jnp.concatenate(
        [v, jnp.full((total - E,), fill, jnp.int32)])


def kernel(x, edge_index, batch, W1, b1, W2, b2, W3, b3, Wf, bf):
    src, dst = edge_index[0], edge_index[1]
    srcp = _padded(src, EPAD, 0).reshape(NW, HALF, 1, HCHW, CH)
    dstp = _padded(dst, EPAD, DUMMY).reshape(NW, HALF, 1, HCHW, CH)
    idxp = jnp.concatenate([srcp, dstp], axis=2)
    dstpd = _padded(dst, EPADD, DUMMY).reshape(NW, CHWD, CHD)

    ones_rows = jnp.ones((CHD, D), jnp.float32)
    zrows_d = jnp.zeros((CHD, D), jnp.float32)
    zrows = jnp.zeros((CH, D), jnp.float32)

    degp = _sc_deg(dstpd, ones_rows, zrows_d)
    degcol = degp[:, :N, 0:1]

    g1 = _m1(x, W1, degcol)
    agg1 = _sc_agg(g1, idxp, zrows)
    g2 = _mlayer(agg1, g1, degcol, b1, W2)
    agg2 = _sc_agg(g2, idxp, zrows)
    g3 = _mlayer(agg2, g2, degcol, b2, W3)
    agg3 = _sc_agg(g3, idxp, zrows)
    return _pool(agg3, g3, degcol, b3, batch, Wf, bf)


# speedup vs baseline: 1.1786x; 1.1786x over previous
import functools

import jax
import jax.numpy as jnp
from jax import lax
from jax.experimental import pallas as pl
from jax.experimental.pallas import tpu as pltpu
from jax.experimental.pallas import tpu_sc as plsc

N = 10000
E = 320000
D = 128
C = 2
G = 16

NC = 2
NSUB = 16
NW = NC * NSUB
NBUF = 2

CHD = 128
CHWD = 80
EPADD = NW * CHWD * CHD

CH = 96
SCH = 36
NST0 = 5
NST1 = 1
TOTST = NSUB * (NST0 + NST1)
EPAD = TOTST * SCH * CH

NP = 10112
TPT = NP // NSUB
DUMMY = N

ROWS = 1000

def _make_splits(step):
    out, off = [], 0
    while off < TPT:
        out.append((off, min(step, TPT - off)))
        off += step
    return out

_SPLITS = _make_splits(CH)
_SPLITS_D = _make_splits(CHD)


def _sc_mesh():
    return plsc.VectorSubcoreMesh(core_axis_name="c", subcore_axis_name="s")


@functools.partial(
    pl.kernel,
    out_type=jax.ShapeDtypeStruct((NC, NP, D), jnp.float32),
    mesh=_sc_mesh(),
    scratch_types=[
        pltpu.VMEM((CHWD, CHD), jnp.int32),
        pltpu.VMEM((CHD, D), jnp.float32),
        pltpu.VMEM((CHD, D), jnp.float32),
        pltpu.VMEM_SHARED((NP, D), jnp.float32),
    ],
)
def _sc_deg(dstp_hbm, ones_hbm, zeros_hbm, out_hbm, dstw, ones_v, z_v, acc):
    c = lax.axis_index("c")
    s = lax.axis_index("s")
    wid = c * NSUB + s
    pltpu.sync_copy(dstp_hbm.at[wid], dstw)
    pltpu.sync_copy(ones_hbm, ones_v)
    pltpu.sync_copy(zeros_hbm, z_v)
    for off, sz in _SPLITS_D:
        pltpu.sync_copy(z_v.at[pl.ds(0, sz)],
                        acc.at[pl.ds(s * TPT + off, sz)])
    plsc.subcore_barrier()

    def chunk(j, carry):
        pltpu.sync_copy(ones_v, acc.at[dstw.at[j]], add=True)
        return carry

    lax.fori_loop(0, CHWD, chunk, 0)
    plsc.subcore_barrier()
    for off, sz in _SPLITS_D:
        pltpu.sync_copy(acc.at[pl.ds(s * TPT + off, sz)],
                        z_v.at[pl.ds(0, sz)])
        pltpu.sync_copy(z_v.at[pl.ds(0, sz)],
                        out_hbm.at[c, pl.ds(s * TPT + off, sz)])


@functools.partial(
    pl.kernel,
    out_type=jax.ShapeDtypeStruct((NC, NP, D), jnp.float32),
    mesh=_sc_mesh(),
    scratch_types=[
        pltpu.VMEM((2, SCH, CH), jnp.int32),
        pltpu.VMEM((CH, D), jnp.float32),
        pltpu.VMEM((CH, D), jnp.float32),
        pltpu.VMEM_SHARED((NP, D), jnp.float32),
        pltpu.SemaphoreType.DMA,
        pltpu.SemaphoreType.DMA,
    ],
)
def _sc_agg(g_hbm, idxp_hbm, zrows_hbm, out_hbm,
            idxw, rows0, rows1, acc, sem0, sem1):
    rows = (rows0, rows1)
    sems = (sem0, sem1)
    srcw = idxw.at[0]
    dstw = idxw.at[1]
    c = lax.axis_index("c")
    s = lax.axis_index("s")
    pltpu.sync_copy(zrows_hbm, rows[0])
    for off, sz in _SPLITS:
        pltpu.sync_copy(rows[0].at[pl.ds(0, sz)],
                        acc.at[pl.ds(s * TPT + off, sz)])
    plsc.subcore_barrier()

    nst = jnp.where(c == 0, NST0, NST1)
    base_st = jnp.where(c == 0, s * NST0, NSUB * NST0 + s * NST1)

    def stage(h, carry):
        pltpu.sync_copy(idxp_hbm.at[base_st + h], idxw)
        for b in range(NBUF):
            pltpu.async_copy(g_hbm.at[srcw.at[b]], rows[b], sems[b])

        def body(i, carry2):
            base = i * NBUF
            for b in range(NBUF):
                e = base + b
                pltpu.make_async_copy(
                    g_hbm.at[srcw.at[e]], rows[b], sems[b]).wait()
                pltpu.sync_copy(rows[b], acc.at[dstw.at[e]], add=True)
                ne = e + NBUF

                @pl.when(ne < SCH)
                def _():
                    pltpu.async_copy(
                        g_hbm.at[srcw.at[ne]], rows[b], sems[b])
            return carry2

        lax.fori_loop(0, SCH // NBUF, body, 0)
        return carry

    lax.fori_loop(0, nst, stage, 0)
    plsc.subcore_barrier()
    for off, sz in _SPLITS:
        pltpu.sync_copy(acc.at[pl.ds(s * TPT + off, sz)],
                        rows[0].at[pl.ds(0, sz)])
        pltpu.sync_copy(rows[0].at[pl.ds(0, sz)],
                        out_hbm.at[c, pl.ds(s * TPT + off, sz)])


def _dinv_of(degcol):
    return lax.rsqrt(degcol[0] + degcol[1] + 1.0)


def _m1_body(x_ref, w_ref, degcol_ref, out_ref):
    dinv = _dinv_of(degcol_ref[...])
    out_ref[...] = jnp.dot(x_ref[...], w_ref[...],
                           preferred_element_type=jnp.float32) * dinv


def _mlayer_body(aggp_ref, gprev_ref, degcol_ref, b_ref, w_ref, out_ref):
    dinv = _dinv_of(degcol_ref[...])
    a = dinv * (aggp_ref[0] + aggp_ref[1] + gprev_ref[...]) + b_ref[...]
    a = jnp.maximum(a, 0.0)
    out_ref[...] = jnp.dot(a, w_ref[...],
                           preferred_element_type=jnp.float32) * dinv


_GSPEC = [
    pl.BlockSpec((NC, ROWS, D), lambda i: (0, i, 0)),
    pl.BlockSpec((ROWS, D), lambda i: (i, 0)),
    pl.BlockSpec((NC, ROWS, 1), lambda i: (0, i, 0)),
    pl.BlockSpec((1, D), lambda i: (0, 0)),
    pl.BlockSpec((D, D), lambda i: (0, 0)),
]


def _m1(x, W, degcol):
    return pl.pallas_call(
        _m1_body,
        grid=(N // ROWS,),
        in_specs=[
            pl.BlockSpec((ROWS, D), lambda i: (i, 0)),
            pl.BlockSpec((D, D), lambda i: (0, 0)),
            pl.BlockSpec((NC, ROWS, 1), lambda i: (0, i, 0)),
        ],
        out_specs=pl.BlockSpec((ROWS, D), lambda i: (i, 0)),
        out_shape=jax.ShapeDtypeStruct((N, D), jnp.float32),
    )(x, W, degcol)


def _mlayer(aggp, gprev, degcol, b, W):
    return pl.pallas_call(
        _mlayer_body,
        grid=(N // ROWS,),
        in_specs=_GSPEC,
        out_specs=pl.BlockSpec((ROWS, D), lambda i: (i, 0)),
        out_shape=jax.ShapeDtypeStruct((N, D), jnp.float32),
    )(aggp, gprev, degcol, b.reshape(1, D), W)


def _pool_body(aggp_ref, g_ref, degcol_ref, b_ref, ids_ref, wf_ref, bf_ref,
               out_ref, scr_ref):
    i = pl.program_id(0)

    @pl.when(i == 0)
    def _():
        scr_ref[...] = jnp.full((G, D), -jnp.inf, jnp.float32)

    dinv = _dinv_of(degcol_ref[...])
    post = dinv * (aggp_ref[0] + aggp_ref[1] + g_ref[...]) + b_ref[...]
    ids = ids_ref[...]
    cur = scr_ref[...]
    maxes = jnp.stack(
        [jnp.max(jnp.where(ids == float(g), post, -jnp.inf), axis=0)
         for g in range(G)])
    cur = jnp.maximum(cur, maxes)
    scr_ref[...] = cur

    @pl.when(i == pl.num_programs(0) - 1)
    def _():
        pooled = jnp.where(jnp.isfinite(cur), cur, 0.0)
        out_ref[...] = (
            jnp.dot(pooled, wf_ref[...], preferred_element_type=jnp.float32)
            + bf_ref[...])


def _pool(aggp, g3, degcol, b3, batch, Wf, bf):
    ids = batch.astype(jnp.float32).reshape(N, 1)
    return pl.pallas_call(
        _pool_body,
        grid=(N // ROWS,),
        in_specs=_GSPEC[:4] + [
            pl.BlockSpec((ROWS, 1), lambda i: (i, 0)),
            pl.BlockSpec((D, C), lambda i: (0, 0)),
            pl.BlockSpec((1, C), lambda i: (0, 0)),
        ],
        out_specs=pl.BlockSpec((G, C), lambda i: (0, 0)),
        out_shape=jax.ShapeDtypeStruct((G, C), jnp.float32),
        scratch_shapes=[pltpu.VMEM((G, D), jnp.float32)],
    )(aggp, g3, degcol, b3.reshape(1, D), ids, Wf, bf.reshape(1, C))


def _padded(v, total, fill):
    return jnp.concatenate(
        [v, jnp.full((total - E,), fill, jnp.int32)])


def kernel(x, edge_index, batch, W1, b1, W2, b2, W3, b3, Wf, bf):
    src, dst = edge_index[0], edge_index[1]
    srcp = _padded(src, EPAD, 0).reshape(TOTST, 1, SCH, CH)
    dstp = _padded(dst, EPAD, DUMMY).reshape(TOTST, 1, SCH, CH)
    idxp = jnp.concatenate([srcp, dstp], axis=1)
    dstpd = _padded(dst, EPADD, DUMMY).reshape(NW, CHWD, CHD)

    ones_rows = jnp.ones((CHD, D), jnp.float32)
    zrows_d = jnp.zeros((CHD, D), jnp.float32)
    zrows = jnp.zeros((CH, D), jnp.float32)

    degp = _sc_deg(dstpd, ones_rows, zrows_d)
    degcol = degp[:, :N, 0:1]

    g1 = _m1(x, W1, degcol)
    agg1 = _sc_agg(g1, idxp, zrows)
    g2 = _mlayer(agg1, g1, degcol, b1, W2)
    agg2 = _sc_agg(g2, idxp, zrows)
    g3 = _mlayer(agg2, g2, degcol, b2, W3)
    agg3 = _sc_agg(g3, idxp, zrows)
    return _pool(agg3, g3, degcol, b3, batch, Wf, bf)
